# baseline (speedup 1.0000x reference)
import jax
import jax.numpy as jnp
from jax import lax
from jax.experimental import pallas as pl
from jax.experimental.pallas import tpu as pltpu
from jax.experimental.pallas import tpu_sc as plsc

_NUM_MAPPERS = 1000000
_EMBED_DIM = 64
_PAD_DIM = 128
_BATCH = 16384
_TRASH = _BATCH

_NC = 2
_NS = 16
_NCOLS = 7813
_CPW = 245
_CCH = 2
_CHW = _CCH * 128
_NCH = 124
_MAXC = _NCOLS - _CCH
_CAP = _BATCH + 16


def _rsqrt(x):
    i = lax.bitcast_convert_type(x, jnp.int32)
    i = jnp.int32(0x5F3759DF) - lax.shift_right_arithmetic(i, 1)
    y = lax.bitcast_convert_type(i, jnp.float32)
    half = jnp.float32(0.5) * x
    for _ in range(3):
        y = y * (jnp.float32(1.5) - half * y * y)
    return y


def _embed_body(ids_hbm, tab3_hbm, gamma_hbm, beta_hbm, out2_hbm,
                ids_v, selid_v, selpos_v, hitid_v, hitpos_v,
                stage_a, stage_b, bank_v, posb_v, gamma_v, beta_v,
                sem, osem):
    wid = lax.axis_index("s") * _NC + lax.axis_index("c")
    lanes = lax.iota(jnp.int32, 16)

    pltpu.sync_copy(ids_hbm, ids_v)
    pltpu.sync_copy(gamma_hbm, gamma_v)
    pltpu.sync_copy(beta_hbm, beta_v)

    lo_col = wid * _CPW
    hi_col = jnp.minimum(lo_col + _CPW, jnp.int32(_NCOLS))
    lo = lo_col * jnp.int32(128)
    hi = hi_col * jnp.int32(128)

    def sel_body(i, off):
        v = ids_v[pl.ds(i * 16, 16)]
        v = jnp.where(v == jnp.int32(-1), jnp.int32(_NUM_MAPPERS), v)
        v = jnp.minimum(jnp.maximum(v, jnp.int32(0)),
                        jnp.int32(_NUM_MAPPERS))
        m = (v >= lo) & (v < hi)
        plsc.store_compressed(selid_v.at[pl.ds(off, 16)], v, mask=m)
        plsc.store_compressed(selpos_v.at[pl.ds(off, 16)],
                              i * 16 + lanes, mask=m)
        return off + plsc.all_reduce_population_count(m)[0]

    total = lax.fori_loop(0, _BATCH // 16, sel_body, jnp.int32(0))
    nvec = (total + jnp.int32(15)) // jnp.int32(16)

    g_vec = [gamma_v[pl.ds(16 * q, 16)] for q in range(4)]
    b_vec = [beta_v[pl.ds(16 * q, 16)] for q in range(4)]
    inv_d = jnp.float32(1.0 / _EMBED_DIM)
    eps = jnp.float32(1e-5)

    for b in range(2):
        posb_v[b, pl.ds(0, 16)] = jnp.broadcast_to(jnp.int32(_TRASH), (16,))
        pltpu.async_copy(bank_v.at[b], out2_hbm.at[posb_v.at[b]], osem)

    def chunk_base(j):
        return (jnp.minimum(lo_col + _CCH * j, jnp.int32(_MAXC))
                * jnp.int32(128))

    def fire(j, buf):
        cb = pl.multiple_of(chunk_base(j), 128)
        return pltpu.async_copy(tab3_hbm.at[:, :, pl.ds(cb, _CHW)], buf,
                                sem)

    def drain(buf):
        pltpu.make_async_copy(tab3_hbm.at[:, :, pl.ds(0, _CHW)],
                              buf, sem).wait()

    def gath(buf, a, b, local):
        return plsc.load_gather(
            buf, [jnp.broadcast_to(jnp.int32(a), (16,)),
                  jnp.broadcast_to(jnp.int32(b), (16,)), local])

    def process(j, buf, g_fired):
        cb = chunk_base(j)

        def scan_body(s, hoff):
            selv = selid_v[pl.ds(s * 16, 16)]
            pv = selpos_v[pl.ds(s * 16, 16)]
            m = (((s * 16 + lanes) < total)
                 & (selv >= cb) & (selv < cb + jnp.int32(_CHW)))
            plsc.store_compressed(hitid_v.at[pl.ds(hoff, 16)], selv,
                                  mask=m)
            plsc.store_compressed(hitpos_v.at[pl.ds(hoff, 16)], pv,
                                  mask=m)
            return hoff + plsc.all_reduce_population_count(m)[0]

        nh = lax.fori_loop(0, nvec, scan_body, jnp.int32(0))
        hvec = (nh + jnp.int32(15)) // jnp.int32(16)

        def hit_body(h, g):
            slot = g % jnp.int32(2)
            pltpu.make_async_copy(bank_v.at[0],
                                  out2_hbm.at[posb_v.at[0]], osem).wait()
            hid = hitid_v[pl.ds(h * 16, 16)]
            hpv = hitpos_v[pl.ds(h * 16, 16)]
            live = (h * 16 + lanes) < nh
            local = jnp.minimum(
                jnp.maximum(hid - cb, jnp.int32(0)), jnp.int32(_CHW - 1))
            slot16 = jnp.broadcast_to(slot, (16,))
            acc_s = jnp.zeros((16,), jnp.float32)
            acc_q = jnp.zeros((16,), jnp.float32)
            cols = []
            for f in range(_EMBED_DIM):
                gv = gath(buf, f // 8, f % 8, local)
                cols.append(gv)
                acc_s = acc_s + gv
                acc_q = acc_q + gv * gv
            mean = acc_s * inv_d
            var = acc_q * inv_d - mean * mean
            rv = _rsqrt(var + eps)
            for f in range(_EMBED_DIM):
                gf = g_vec[f // 16][f % 16]
                bf = b_vec[f // 16][f % 16]
                n = (cols[f] - mean) * rv * gf + bf
                plsc.store_scatter(
                    bank_v,
                    [slot16, lanes,
                     jnp.broadcast_to(jnp.int32(f), (16,))], n)
            plsc.store_scatter(
                posb_v, [slot16, lanes],
                jnp.where(live, hpv, jnp.int32(_TRASH)))
            pltpu.async_copy(bank_v.at[slot], out2_hbm.at[posb_v.at[slot]],
                             osem)
            return g + jnp.int32(1)

        return lax.fori_loop(0, hvec, hit_body, g_fired)

    fire(0, stage_a)
    fire(1, stage_b)

    def pair_body(jj, g):
        ja = jj * 2
        drain(stage_a)
        g = process(ja, stage_a, g)
        fire(ja + 2, stage_a)
        drain(stage_b)
        g = process(ja + 1, stage_b, g)
        fire(ja + 3, stage_b)
        return g

    lax.fori_loop(0, _NCH // 2, pair_body, jnp.int32(0))
    drain(stage_a)
    drain(stage_b)
    for b in range(2):
        pltpu.make_async_copy(bank_v.at[0], out2_hbm.at[posb_v.at[0]],
                              osem).wait()


@jax.jit
def _embed(mapper_ids, table, ln_gamma, ln_beta):
    mesh = plsc.VectorSubcoreMesh(core_axis_name="c", subcore_axis_name="s")
    f = pl.kernel(
        _embed_body,
        mesh=mesh,
        compiler_params=pltpu.CompilerParams(
            use_tc_tiling_on_sc=True, needs_layout_passes=False),
        out_type=jax.ShapeDtypeStruct((_BATCH + 1, _PAD_DIM), jnp.float32),
        scratch_types=[
            pltpu.VMEM((_BATCH,), jnp.int32),
            pltpu.VMEM((_CAP,), jnp.int32),
            pltpu.VMEM((_CAP,), jnp.int32),
            pltpu.VMEM((_CAP,), jnp.int32),
            pltpu.VMEM((_CAP,), jnp.int32),
            pltpu.VMEM((8, 8, _CHW), jnp.float32),
            pltpu.VMEM((8, 8, _CHW), jnp.float32),
            pltpu.VMEM((2, 16, _PAD_DIM), jnp.float32),
            pltpu.VMEM((2, 16), jnp.int32),
            pltpu.VMEM((_EMBED_DIM,), jnp.float32),
            pltpu.VMEM((_EMBED_DIM,), jnp.float32),
            pltpu.SemaphoreType.DMA,
            pltpu.SemaphoreType.DMA,
        ],
    )
    tab3 = table.T.reshape(8, 8, _NUM_MAPPERS + 1)
    out2 = f(mapper_ids, tab3, ln_gamma, ln_beta)
    return out2[:_BATCH, :_EMBED_DIM]


def kernel(mapper_ids, table, ln_gamma, ln_beta):
    return _embed(mapper_ids, table, ln_gamma, ln_beta)


# speedup vs baseline: 2.2050x; 2.2050x over previous
import jax
import jax.numpy as jnp
from jax import lax
from jax.experimental import pallas as pl
from jax.experimental.pallas import tpu as pltpu
from jax.experimental.pallas import tpu_sc as plsc

_NUM_MAPPERS = 1000000
_EMBED_DIM = 64
_PAD_DIM = 128
_BATCH = 16384
_TRASH = _BATCH

_NC = 2
_NS = 16
_NCOLS = 7813
_CPW = 245
_CCH = 4
_CHW = _CCH * 128
_NCH = 62
_MAXC = _NCOLS - _CCH
_CAP = _BATCH + 16


def _rsqrt(x):
    i = lax.bitcast_convert_type(x, jnp.int32)
    i = jnp.int32(0x5F3759DF) - lax.shift_right_arithmetic(i, 1)
    y = lax.bitcast_convert_type(i, jnp.float32)
    half = jnp.float32(0.5) * x
    for _ in range(3):
        y = y * (jnp.float32(1.5) - half * y * y)
    return y


def _embed_body(ids_hbm, tab3_hbm, gamma_hbm, beta_hbm, out2_hbm,
                ids_v, selid_v, hit_v,
                stage_a, stage_b, bank_v, posb_v, gamma_v, beta_v,
                sem, osem):
    wid = lax.axis_index("s") * _NC + lax.axis_index("c")
    lanes = lax.iota(jnp.int32, 16)

    pltpu.sync_copy(ids_hbm, ids_v)
    pltpu.sync_copy(gamma_hbm, gamma_v)
    pltpu.sync_copy(beta_hbm, beta_v)

    lo_col = wid * _CPW
    hi_col = jnp.minimum(lo_col + _CPW, jnp.int32(_NCOLS))
    lo = lo_col * jnp.int32(128)
    hi = hi_col * jnp.int32(128)

    def sel_body(i, off):
        v = ids_v[pl.ds(i * 16, 16)]
        v = jnp.where(v == jnp.int32(-1), jnp.int32(_NUM_MAPPERS), v)
        v = jnp.minimum(jnp.maximum(v, jnp.int32(0)),
                        jnp.int32(_NUM_MAPPERS))
        m = (v >= lo) & (v < hi)
        packed = lax.shift_left(i * 16 + lanes, jnp.int32(15)) | (v - lo)
        plsc.store_compressed(selid_v.at[pl.ds(off, 16)], packed, mask=m)
        return off + plsc.all_reduce_population_count(m)[0]

    total = lax.fori_loop(0, _BATCH // 16, sel_body, jnp.int32(0))
    nvec = (total + jnp.int32(15)) // jnp.int32(16)

    g_vec = [gamma_v[pl.ds(16 * q, 16)] for q in range(4)]
    b_vec = [beta_v[pl.ds(16 * q, 16)] for q in range(4)]
    inv_d = jnp.float32(1.0 / _EMBED_DIM)
    eps = jnp.float32(1e-5)

    for b in range(2):
        posb_v[b, pl.ds(0, 16)] = jnp.broadcast_to(jnp.int32(_TRASH), (16,))
        pltpu.async_copy(bank_v.at[b], out2_hbm.at[posb_v.at[b]], osem)

    def chunk_base(j):
        return (jnp.minimum(lo_col + _CCH * j, jnp.int32(_MAXC))
                * jnp.int32(128))

    def fire(j, buf):
        cb = pl.multiple_of(chunk_base(j), 128)
        return pltpu.async_copy(tab3_hbm.at[:, :, pl.ds(cb, _CHW)], buf,
                                sem)

    def drain(buf):
        pltpu.make_async_copy(tab3_hbm.at[:, :, pl.ds(0, _CHW)],
                              buf, sem).wait()

    def gath(buf, a, b, local):
        return plsc.load_gather(
            buf, [jnp.broadcast_to(jnp.int32(a), (16,)),
                  jnp.broadcast_to(jnp.int32(b), (16,)), local])

    def process(j, buf, g_fired):
        cb = chunk_base(j)

        cbr = cb - lo

        def scan_body(s, hoff):
            spk = selid_v[pl.ds(s * 16, 16)]
            rel = spk & jnp.int32(0x7FFF)
            pv = lax.shift_right_logical(spk, jnp.int32(15))
            m = (((s * 16 + lanes) < total)
                 & (rel >= cbr) & (rel < cbr + jnp.int32(_CHW)))
            packed = lax.shift_left(pv, jnp.int32(9)) | (rel - cbr)
            plsc.store_compressed(hit_v.at[pl.ds(hoff, 16)], packed,
                                  mask=m)
            return hoff + plsc.all_reduce_population_count(m)[0]

        nh = lax.fori_loop(0, nvec, scan_body, jnp.int32(0))
        hvec = (nh + jnp.int32(15)) // jnp.int32(16)

        def hit_body(h, g):
            slot = g % jnp.int32(2)
            pltpu.make_async_copy(bank_v.at[0],
                                  out2_hbm.at[posb_v.at[0]], osem).wait()
            hpk = hit_v[pl.ds(h * 16, 16)]
            hpv = lax.shift_right_logical(hpk, jnp.int32(9))
            live = (h * 16 + lanes) < nh
            local = jnp.minimum(
                jnp.maximum(hpk & jnp.int32(_CHW - 1), jnp.int32(0)),
                jnp.int32(_CHW - 1))
            slot16 = jnp.broadcast_to(slot, (16,))
            acc_s = jnp.zeros((16,), jnp.float32)
            acc_q = jnp.zeros((16,), jnp.float32)
            cols = []
            for f in range(_EMBED_DIM):
                gv = gath(buf, f // 8, f % 8, local)
                cols.append(gv)
                acc_s = acc_s + gv
                acc_q = acc_q + gv * gv
            mean = acc_s * inv_d
            var = acc_q * inv_d - mean * mean
            rv = _rsqrt(var + eps)
            for f in range(_EMBED_DIM):
                gf = g_vec[f // 16][f % 16]
                bf = b_vec[f // 16][f % 16]
                n = (cols[f] - mean) * rv * gf + bf
                plsc.store_scatter(
                    bank_v,
                    [slot16, lanes,
                     jnp.broadcast_to(jnp.int32(f), (16,))], n)
            plsc.store_scatter(
                posb_v, [slot16, lanes],
                jnp.where(live, hpv, jnp.int32(_TRASH)))
            pltpu.async_copy(bank_v.at[slot], out2_hbm.at[posb_v.at[slot]],
                             osem)
            return g + jnp.int32(1)

        return lax.fori_loop(0, hvec, hit_body, g_fired)

    fire(0, stage_a)
    fire(1, stage_b)

    def pair_body(jj, g):
        ja = jj * 2
        drain(stage_a)
        g = process(ja, stage_a, g)
        fire(ja + 2, stage_a)
        drain(stage_b)
        g = process(ja + 1, stage_b, g)
        fire(ja + 3, stage_b)
        return g

    lax.fori_loop(0, _NCH // 2, pair_body, jnp.int32(0))
    drain(stage_a)
    drain(stage_b)
    for b in range(2):
        pltpu.make_async_copy(bank_v.at[0], out2_hbm.at[posb_v.at[0]],
                              osem).wait()


@jax.jit
def _embed(mapper_ids, table, ln_gamma, ln_beta):
    mesh = plsc.VectorSubcoreMesh(core_axis_name="c", subcore_axis_name="s")
    f = pl.kernel(
        _embed_body,
        mesh=mesh,
        compiler_params=pltpu.CompilerParams(
            use_tc_tiling_on_sc=True, needs_layout_passes=False),
        out_type=jax.ShapeDtypeStruct((_BATCH + 1, _PAD_DIM), jnp.float32),
        scratch_types=[
            pltpu.VMEM((_BATCH,), jnp.int32),
            pltpu.VMEM((_CAP,), jnp.int32),
            pltpu.VMEM((_CAP,), jnp.int32),
            pltpu.VMEM((8, 8, _CHW), jnp.float32),
            pltpu.VMEM((8, 8, _CHW), jnp.float32),
            pltpu.VMEM((2, 16, _PAD_DIM), jnp.float32),
            pltpu.VMEM((2, 16), jnp.int32),
            pltpu.VMEM((_EMBED_DIM,), jnp.float32),
            pltpu.VMEM((_EMBED_DIM,), jnp.float32),
            pltpu.SemaphoreType.DMA,
            pltpu.SemaphoreType.DMA,
        ],
    )
    tab3 = table.T.reshape(8, 8, _NUM_MAPPERS + 1)
    out2 = f(mapper_ids, tab3, ln_gamma, ln_beta)
    return out2[:_BATCH, :_EMBED_DIM]


def kernel(mapper_ids, table, ln_gamma, ln_beta):
    return _embed(mapper_ids, table, ln_gamma, ln_beta)


# speedup vs baseline: 3.8477x; 1.7450x over previous
import jax
import jax.numpy as jnp
from jax import lax
from jax.experimental import pallas as pl
from jax.experimental.pallas import tpu as pltpu
from jax.experimental.pallas import tpu_sc as plsc

_NUM_MAPPERS = 1000000
_EMBED_DIM = 64
_PAD_DIM = 128
_BATCH = 16384

_NC = 2
_NS = 16
_NW = _NC * _NS
_BPW = _BATCH // _NW
_CHUNK = 128
_NJ = _BPW // _CHUNK


def _lane_sum(v):
    lanes = lax.iota(jnp.int32, 16)
    dnums = lax.GatherDimensionNumbers(
        offset_dims=(), collapsed_slice_dims=(0,), start_index_map=(0,))
    for s in (8, 4, 2, 1):
        perm = lax.gather(v, (lanes ^ s)[:, None], dnums, (1,),
                          mode=lax.GatherScatterMode.PROMISE_IN_BOUNDS)
        v = v + perm
    return v


def _rsqrt(x):
    i = lax.bitcast_convert_type(x, jnp.int32)
    i = jnp.int32(0x5F3759DF) - lax.shift_right_arithmetic(i, 1)
    y = lax.bitcast_convert_type(i, jnp.float32)
    half = jnp.float32(0.5) * x
    for _ in range(3):
        y = y * (jnp.float32(1.5) - half * y * y)
    return y


def _embed_body(ids_hbm, table_hbm, gamma_hbm, beta_hbm, out_hbm,
                idx2_v, rows_v, gamma_v, beta_v, sem):
    wid = lax.axis_index("s") * _NC + lax.axis_index("c")
    base = wid * _BPW

    for j in range(_NJ):
        pltpu.sync_copy(ids_hbm.at[pl.ds(base + j * _CHUNK, _CHUNK)],
                        idx2_v.at[j])
    pltpu.sync_copy(gamma_hbm, gamma_v)
    pltpu.sync_copy(beta_hbm, beta_v)

    for j in range(_NJ):
        for i in range(_CHUNK // 16):
            v = idx2_v[j, pl.ds(i * 16, 16)]
            v = jnp.where(v == jnp.int32(-1), jnp.int32(_NUM_MAPPERS), v)
            v = jnp.minimum(jnp.maximum(v, jnp.int32(0)),
                            jnp.int32(_NUM_MAPPERS))
            idx2_v[j, pl.ds(i * 16, 16)] = v

    copies = []
    for j in range(_NJ):
        copies.append(pltpu.async_copy(
            table_hbm.at[idx2_v.at[j]],
            rows_v.at[pl.ds(j * _CHUNK, _CHUNK), :],
            sem))
    for c in copies:
        c.wait()

    g0 = gamma_v[pl.ds(0, 16)]
    g1 = gamma_v[pl.ds(16, 16)]
    g2 = gamma_v[pl.ds(32, 16)]
    g3 = gamma_v[pl.ds(48, 16)]
    b0 = beta_v[pl.ds(0, 16)]
    b1 = beta_v[pl.ds(16, 16)]
    b2 = beta_v[pl.ds(32, 16)]
    b3 = beta_v[pl.ds(48, 16)]

    inv_d = jnp.float32(1.0 / _EMBED_DIM)
    eps = jnp.float32(1e-5)

    @plsc.parallel_loop(0, _BPW, unroll=4)
    def row_body(r):
        v0 = rows_v[r, pl.ds(0, 16)]
        v1 = rows_v[r, pl.ds(16, 16)]
        v2 = rows_v[r, pl.ds(32, 16)]
        v3 = rows_v[r, pl.ds(48, 16)]
        tot = _lane_sum((v0 + v1) + (v2 + v3))
        mean = tot * inv_d
        t0 = v0 - mean
        t1 = v1 - mean
        t2 = v2 - mean
        t3 = v3 - mean
        sq = _lane_sum((t0 * t0 + t1 * t1) + (t2 * t2 + t3 * t3))
        rv = _rsqrt(sq * inv_d + eps)
        rows_v[r, pl.ds(0, 16)] = t0 * rv * g0 + b0
        rows_v[r, pl.ds(16, 16)] = t1 * rv * g1 + b1
        rows_v[r, pl.ds(32, 16)] = t2 * rv * g2 + b2
        rows_v[r, pl.ds(48, 16)] = t3 * rv * g3 + b3

    pltpu.sync_copy(rows_v, out_hbm.at[pl.ds(base, _BPW), :])


@jax.jit
def _embed(mapper_ids, table, ln_gamma, ln_beta):
    mesh = plsc.VectorSubcoreMesh(core_axis_name="c", subcore_axis_name="s")
    f = pl.kernel(
        _embed_body,
        mesh=mesh,
        compiler_params=pltpu.CompilerParams(
            use_tc_tiling_on_sc=True, needs_layout_passes=False),
        out_type=jax.ShapeDtypeStruct((_BATCH, _PAD_DIM), jnp.float32),
        scratch_types=[
            pltpu.VMEM((_NJ, _CHUNK), jnp.int32),
            pltpu.VMEM((_BPW, _PAD_DIM), jnp.float32),
            pltpu.VMEM((_EMBED_DIM,), jnp.float32),
            pltpu.VMEM((_EMBED_DIM,), jnp.float32),
            pltpu.SemaphoreType.DMA,
        ],
    )
    table_p = jnp.pad(table, ((0, 0), (0, _PAD_DIM - _EMBED_DIM)))
    out_p = f(mapper_ids, table_p, ln_gamma, ln_beta)
    return out_p[:, :_EMBED_DIM]


def kernel(mapper_ids, table, ln_gamma, ln_beta):
    return _embed(mapper_ids, table, ln_gamma, ln_beta)
